# baseline (speedup 1.0000x reference)
import math

import jax
import jax.numpy as jnp
from jax import lax
from jax.experimental import pallas as pl
from jax.experimental.pallas import tpu as pltpu
from jax.experimental.pallas import tpu_sc as plsc

EMB = 32
SCALE = math.sqrt(float(EMB))

NUM_CORES = 2
NUM_SUBCORES = 16
NUM_WORKERS = NUM_CORES * NUM_SUBCORES

CHUNK = 640
NBUF = 4


def _scale_buf(buf):
    @plsc.parallel_loop(0, CHUNK, step=1, unroll=8)
    def _(r):
        buf[r, pl.ds(0, 16)] = buf[r, pl.ds(0, 16)] * SCALE
        buf[r, pl.ds(16, 16)] = buf[r, pl.ds(16, 16)] * SCALE


def _emb_body(n_groups, rows_per_worker, tokens_hbm, table_hbm, out_hbm,
              idx_all, rows, gsems, osems):
    wid = lax.axis_index("s") * NUM_CORES + lax.axis_index("c")
    base = wid * rows_per_worker

    pltpu.sync_copy(tokens_hbm.at[pl.ds(base, rows_per_worker)], idx_all)

    def group(g, carry):
        descs = []
        for b in range(NBUF):
            c = g * NBUF + b
            out_slc = out_hbm.at[pl.ds(base + c * CHUNK, CHUNK)]

            @pl.when(g > 0)
            def _():
                pltpu.make_async_copy(rows[b], out_slc, osems[b]).wait()

            idx_slc = idx_all.at[pl.ds(c * CHUNK, CHUNK)]
            descs.append(
                pltpu.async_copy(table_hbm.at[idx_slc], rows[b], gsems[b]))

        for b in range(NBUF):
            c = g * NBUF + b
            descs[b].wait()
            _scale_buf(rows[b])
            pltpu.async_copy(
                rows[b], out_hbm.at[pl.ds(base + c * CHUNK, CHUNK)], osems[b])
        return carry

    lax.fori_loop(0, n_groups, group, 0)

    for b in range(NBUF):
        pltpu.make_async_copy(
            rows[b], out_hbm.at[pl.ds(base + b * CHUNK, CHUNK)],
            osems[b]).wait()


TR_BLOCK = 8192


def _tc_transpose_body(in_ref, out_ref):
    x = in_ref[...]
    row = jax.lax.broadcasted_iota(jnp.int32, (EMB, EMB), 0)
    col = jax.lax.broadcasted_iota(jnp.int32, (EMB, EMB), 1)
    eye = (row == col).astype(jnp.float32)
    y = jax.lax.dot_general(x, eye, (((0,), (0,)), ((), ())),
                            preferred_element_type=jnp.float32)
    y3 = y.reshape(TR_BLOCK // 4, 4, EMB)
    out_ref[...] = jnp.concatenate([y3[:, j, :] for j in range(4)], axis=1)


def _pack_table(table):
    vocab = table.shape[0]
    table_t = table.T
    packed = pl.pallas_call(
        _tc_transpose_body,
        grid=(pl.cdiv(vocab, TR_BLOCK),),
        in_specs=[pl.BlockSpec((EMB, TR_BLOCK), lambda i: (0, i))],
        out_specs=pl.BlockSpec((TR_BLOCK // 4, 4 * EMB), lambda i: (i, 0)),
        out_shape=jax.ShapeDtypeStruct((vocab // 4, 4 * EMB), jnp.float32),
    )(table_t)
    return packed.reshape(vocab, EMB)


def kernel(tokens, table):
    flat = tokens.T.reshape(-1).astype(jnp.int32)
    table = _pack_table(table)
    n = flat.shape[0]
    assert n % (NUM_WORKERS * CHUNK * NBUF) == 0
    rows_per_worker = n // NUM_WORKERS
    n_groups = rows_per_worker // (CHUNK * NBUF)

    mesh = plsc.VectorSubcoreMesh(core_axis_name="c", subcore_axis_name="s")
    run = pl.kernel(
        lambda t, tb, o, idx, *bufs: _emb_body(
            n_groups, rows_per_worker, t, tb, o, idx,
            list(bufs[:NBUF]), list(bufs[NBUF:2 * NBUF]),
            list(bufs[2 * NBUF:])),
        out_type=jax.ShapeDtypeStruct((n, EMB), jnp.float32),
        mesh=mesh,
        scratch_types=(
            [pltpu.VMEM((n // NUM_WORKERS,), jnp.int32)]
            + [pltpu.VMEM((CHUNK, EMB), jnp.float32) for _ in range(NBUF)]
            + [pltpu.SemaphoreType.DMA for _ in range(2 * NBUF)]
        ),
        compiler_params=pltpu.CompilerParams(use_tc_tiling_on_sc=False),
    )
    out = run(flat, table)
    seq, batch = tokens.shape[1], tokens.shape[0]
    return out.reshape(seq, batch, EMB).transpose(1, 0, 2)


# speedup vs baseline: 1.0312x; 1.0312x over previous
import math

import jax
import jax.numpy as jnp
from jax import lax
from jax.experimental import pallas as pl
from jax.experimental.pallas import tpu as pltpu
from jax.experimental.pallas import tpu_sc as plsc

EMB = 32
SCALE = math.sqrt(float(EMB))

NUM_CORES = 2
NUM_SUBCORES = 16
NUM_WORKERS = NUM_CORES * NUM_SUBCORES

CHUNK = 512
NBUF = 5
BATCH = 4096


def _scale_buf(buf):
    @plsc.parallel_loop(0, CHUNK, step=1, unroll=8)
    def _(r):
        buf[r, pl.ds(0, 16)] = buf[r, pl.ds(0, 16)] * SCALE
        buf[r, pl.ds(16, 16)] = buf[r, pl.ds(16, 16)] * SCALE


def _emb_body(n_groups, rows_per_worker, tokens_hbm, table_hbm, out_hbm,
              idx_all, rows, gsems, osems):
    wid = lax.axis_index("s") * NUM_CORES + lax.axis_index("c")
    base = wid * rows_per_worker

    pltpu.sync_copy(tokens_hbm.at[pl.ds(base, rows_per_worker)], idx_all)

    def out_slice(c):
        start = base + c * CHUNK
        s = start // BATCH
        boff = start % BATCH
        return out_hbm.at[s, pl.ds(boff, CHUNK)]

    def group(g, carry):
        descs = []
        for b in range(NBUF):
            c = g * NBUF + b

            @pl.when(g > 0)
            def _():
                pltpu.make_async_copy(rows[b], out_slice(c), osems[b]).wait()

            idx_slc = idx_all.at[pl.ds(c * CHUNK, CHUNK)]
            descs.append(
                pltpu.async_copy(table_hbm.at[idx_slc], rows[b], gsems[b]))

        for b in range(NBUF):
            c = g * NBUF + b
            descs[b].wait()
            _scale_buf(rows[b])
            pltpu.async_copy(rows[b], out_slice(c), osems[b])
        return carry

    lax.fori_loop(0, n_groups, group, 0)

    for b in range(NBUF):
        pltpu.make_async_copy(rows[b], out_slice(b), osems[b]).wait()


TR_BLOCK = 8192


def _tc_transpose_body(in_ref, out_ref):
    y = in_ref[...].T
    y3 = y.reshape(TR_BLOCK // 4, 4, EMB)
    out_ref[...] = jnp.concatenate([y3[:, j, :] for j in range(4)], axis=1)


def _pack_table(table):
    vocab = table.shape[0]
    table_t = table.T
    packed = pl.pallas_call(
        _tc_transpose_body,
        grid=(pl.cdiv(vocab, TR_BLOCK),),
        in_specs=[pl.BlockSpec((EMB, TR_BLOCK), lambda i: (0, i))],
        out_specs=pl.BlockSpec((TR_BLOCK // 4, 4 * EMB), lambda i: (i, 0)),
        out_shape=jax.ShapeDtypeStruct((vocab // 4, 4 * EMB), jnp.float32),
    )(table_t)
    return packed.reshape(vocab, EMB)


def kernel(tokens, table):
    flat = tokens.T.reshape(-1).astype(jnp.int32)
    table = _pack_table(table)
    n = flat.shape[0]
    assert n % (NUM_WORKERS * CHUNK * NBUF) == 0
    rows_per_worker = n // NUM_WORKERS
    n_groups = rows_per_worker // (CHUNK * NBUF)

    seq, batch = tokens.shape[1], tokens.shape[0]
    mesh = plsc.VectorSubcoreMesh(core_axis_name="c", subcore_axis_name="s")
    run = pl.kernel(
        lambda t, tb, o, idx, *bufs: _emb_body(
            n_groups, rows_per_worker, t, tb, o, idx,
            list(bufs[:NBUF]), list(bufs[NBUF:2 * NBUF]),
            list(bufs[2 * NBUF:])),
        out_type=jax.ShapeDtypeStruct((seq, batch, EMB), jnp.float32),
        mesh=mesh,
        scratch_types=(
            [pltpu.VMEM((n // NUM_WORKERS,), jnp.int32)]
            + [pltpu.VMEM((CHUNK, EMB), jnp.float32) for _ in range(NBUF)]
            + [pltpu.SemaphoreType.DMA for _ in range(2 * NBUF)]
        ),
        compiler_params=pltpu.CompilerParams(use_tc_tiling_on_sc=False),
    )
    out = run(flat, table)
    return out.transpose(1, 0, 2)


# speedup vs baseline: 1.0435x; 1.0120x over previous
import math

import jax
import jax.numpy as jnp
from jax import lax
from jax.experimental import pallas as pl
from jax.experimental.pallas import tpu as pltpu
from jax.experimental.pallas import tpu_sc as plsc

EMB = 32
SCALE = math.sqrt(float(EMB))

NUM_CORES = 2
NUM_SUBCORES = 16
NUM_WORKERS = NUM_CORES * NUM_SUBCORES

CHUNK = 512
NBUF = 5
BATCH = 4096


def _scale_buf(buf):
    @plsc.parallel_loop(0, CHUNK, step=1, unroll=8)
    def _(r):
        buf[r, pl.ds(0, 16)] = buf[r, pl.ds(0, 16)] * SCALE
        buf[r, pl.ds(16, 16)] = buf[r, pl.ds(16, 16)] * SCALE


def _emb_body(n_groups, rows_per_worker, tokens_hbm, table_hbm, out_hbm,
              idx_all, rows, gsems, osems):
    wid = lax.axis_index("s") * NUM_CORES + lax.axis_index("c")
    base = wid * rows_per_worker

    pltpu.sync_copy(tokens_hbm.at[pl.ds(base, rows_per_worker)], idx_all)

    def out_slice(c):
        start = base + c * CHUNK
        s = start // BATCH
        boff = start % BATCH
        return out_hbm.at[s, pl.ds(boff, CHUNK)]

    def group(g, carry):
        descs = []
        for b in range(NBUF):
            c = g * NBUF + b

            @pl.when(g > 0)
            def _():
                pltpu.make_async_copy(rows[b], out_slice(c), osems[b]).wait()

            idx_slc = idx_all.at[pl.ds(c * CHUNK, CHUNK)]
            descs.append(
                pltpu.async_copy(table_hbm.at[idx_slc], rows[b], gsems[b]))

        for b in range(NBUF):
            c = g * NBUF + b
            descs[b].wait()
            _scale_buf(rows[b])
            pltpu.async_copy(rows[b], out_slice(c), osems[b])
        return carry

    lax.fori_loop(0, n_groups, group, 0)

    for b in range(NBUF):
        pltpu.make_async_copy(rows[b], out_slice(b), osems[b]).wait()


TR_BLOCK = 32768


def _tc_transpose_body(in_ref, out_ref):
    y = in_ref[...].T
    y3 = y.reshape(TR_BLOCK // 4, 4, EMB)
    out_ref[...] = jnp.concatenate([y3[:, j, :] for j in range(4)], axis=1)


def _pack_table(table):
    vocab = table.shape[0]
    table_t = table.T
    packed = pl.pallas_call(
        _tc_transpose_body,
        grid=(pl.cdiv(vocab, TR_BLOCK),),
        in_specs=[pl.BlockSpec((EMB, TR_BLOCK), lambda i: (0, i))],
        out_specs=pl.BlockSpec((TR_BLOCK // 4, 4 * EMB), lambda i: (i, 0)),
        out_shape=jax.ShapeDtypeStruct((vocab // 4, 4 * EMB), jnp.float32),
    )(table_t)
    return packed.reshape(vocab, EMB)


def kernel(tokens, table):
    flat = tokens.T.reshape(-1).astype(jnp.int32)
    table = _pack_table(table)
    n = flat.shape[0]
    assert n % (NUM_WORKERS * CHUNK * NBUF) == 0
    rows_per_worker = n // NUM_WORKERS
    n_groups = rows_per_worker // (CHUNK * NBUF)

    seq, batch = tokens.shape[1], tokens.shape[0]
    mesh = plsc.VectorSubcoreMesh(core_axis_name="c", subcore_axis_name="s")
    run = pl.kernel(
        lambda t, tb, o, idx, *bufs: _emb_body(
            n_groups, rows_per_worker, t, tb, o, idx,
            list(bufs[:NBUF]), list(bufs[NBUF:2 * NBUF]),
            list(bufs[2 * NBUF:])),
        out_type=jax.ShapeDtypeStruct((seq, batch, EMB), jnp.float32),
        mesh=mesh,
        scratch_types=(
            [pltpu.VMEM((n // NUM_WORKERS,), jnp.int32)]
            + [pltpu.VMEM((CHUNK, EMB), jnp.float32) for _ in range(NBUF)]
            + [pltpu.SemaphoreType.DMA for _ in range(2 * NBUF)]
        ),
        compiler_params=pltpu.CompilerParams(use_tc_tiling_on_sc=False),
    )
    out = run(flat, table)
    return out.transpose(1, 0, 2)
